# Initial kernel scaffold; baseline (speedup 1.0000x reference)
import jax
import jax.numpy as jnp
from jax.experimental import pallas as pl


def kernel(X, edge_index, edge_weight, W1, b1, g1, be1, W2, b2, g2, be2, Wih1, Whh1, bih1, bhh1, Wih2, Whh2, bih2, bhh2):
    raise NotImplementedError("write your pallas kernel here")


# speedup vs baseline: 8.9406x; 8.9406x over previous
import functools

import jax
import jax.numpy as jnp
from jax import lax
from jax.experimental import pallas as pl
from jax.experimental.pallas import tpu as pltpu
from jax.experimental.pallas import tpu_sc as plsc

WINDOW = 4
NUM_NODES = 10000
IN_CH = 128
HID = 128
NTOT = WINDOW * NUM_NODES
E = 600000
EPS = 1e-5

NC = 2
NS = 16
NTILES = NC * NS

PT_ROWS = 152
PT = PT_ROWS * 128
EPAD = NTILES * PT

NCHUNK = 4
CH = NUM_NODES
PAD_ROWS = 368
ACC_ROWS = CH + PAD_ROWS
ZROWS = 216
EW = 8
NWIN = PT_ROWS // EW
GW = 128

_mesh = plsc.VectorSubcoreMesh(core_axis_name="c", subcore_axis_name="s")


def _zero_vmem(buf, rows):
    z = jnp.zeros((16,), jnp.float32)

    def body(r, _):
        for k in range(8):
            buf[r, pl.ds(k * 16, 16)] = z
        return 0

    lax.fori_loop(0, rows, body, 0, unroll=2)


def _sc_deg_body(col_hbm, w_hbm, out_hbm, col_v, w_v, zrow, acc, sem):
    cid = lax.axis_index("c")
    sid = lax.axis_index("s")
    wid = cid * NS + sid

    z = jnp.zeros((16,), jnp.float32)

    def zb(r, _):
        zrow[pl.ds(r * 16, 16)] = z
        return 0

    lax.fori_loop(0, 160, zb, 0)
    pltpu.sync_copy(zrow.at[pl.ds(0, 2496)], acc.at[pl.ds(sid * 2496, 2496)])

    @pl.when(sid == NS - 1)
    def _():
        pltpu.sync_copy(zrow.at[pl.ds(0, 64)], acc.at[pl.ds(39936, 64)])

    plsc.subcore_barrier()

    pltpu.sync_copy(col_hbm.at[wid], col_v)
    pltpu.sync_copy(w_hbm.at[wid], w_v)

    def body(j, _):
        pltpu.sync_copy(w_v.at[j], acc.at[col_v.at[j]], add=True)
        return 0

    lax.fori_loop(0, PT_ROWS, body, 0)
    plsc.subcore_barrier()
    pltpu.sync_copy(acc.at[pl.ds(sid * 2496, 2496)], zrow.at[pl.ds(0, 2496)])
    pltpu.sync_copy(zrow.at[pl.ds(0, 2496)],
                    out_hbm.at[pl.ds(cid * NTOT + sid * 2496, 2496)])

    @pl.when(sid == NS - 1)
    def _():
        pltpu.sync_copy(acc.at[pl.ds(39936, 64)], zrow.at[pl.ds(0, 64)])
        pltpu.sync_copy(zrow.at[pl.ds(0, 64)],
                        out_hbm.at[pl.ds(cid * NTOT + 39936, 64)])


_sc_deg = pl.kernel(
    _sc_deg_body,
    out_type=jax.ShapeDtypeStruct((NC * NTOT,), jnp.float32),
    mesh=_mesh,
    scratch_types=[
        pltpu.VMEM((PT_ROWS, 128), jnp.int32),
        pltpu.VMEM((PT_ROWS, 128), jnp.float32),
        pltpu.VMEM((2560,), jnp.float32),
        pltpu.VMEM_SHARED((NTOT,), jnp.float32),
        pltpu.SemaphoreType.DMA,
    ],
)


def _sc_scatter_body(xs_hbm, row_hbm, col_hbm, w_hbm, out_hbm,
                     row_v, col_v, w_v, lcol_v, weff_v,
                     rb0, rb1, acc, sem, sem_g0, sem_g1):
    cid = lax.axis_index("c")
    sid = lax.axis_index("s")
    iota = lax.iota(jnp.int32, 16)
    dummy_base = CH + sid * 23

    def gstart(j, buf, gsem):
        return pltpu.async_copy(xs_hbm.at[row_v.at[j]], buf, gsem)

    def gwait(buf, gsem):
        pltpu.make_async_copy(xs_hbm.at[row_v.at[0]], buf, gsem).wait()

    def process(j, rb):
        def mul(q, _):
            wvec = weff_v[j, pl.ds(q * 16, 16)]
            for e in range(16):
                ws = wvec[e]
                r2 = q * 16 + e
                for k in range(8):
                    sl = pl.ds(k * 16, 16)
                    rb[r2, sl] = rb[r2, sl] * ws
            return 0

        lax.fori_loop(0, GW // 16, mul, 0)
        pltpu.sync_copy(rb, acc.at[lcol_v.at[j]], add=True)

    def chunk_body(hi, _):
        lo = (cid * 2 + hi) * CH

        _zero_vmem(rb0, GW)
        for p in range(5):
            pltpu.sync_copy(rb0, acc.at[pl.ds(sid * 648 + p * GW, GW)])
        pltpu.sync_copy(rb0.at[pl.ds(0, 8)],
                        acc.at[pl.ds(sid * 648 + 640, 8)])
        plsc.subcore_barrier()

        def group_body(gi, _):
            g = sid * 2 + gi

            def window(wi, _):
                pltpu.sync_copy(row_hbm.at[g, pl.ds(wi * EW, EW)], row_v)
                pltpu.sync_copy(col_hbm.at[g, pl.ds(wi * EW, EW)], col_v)
                pltpu.sync_copy(w_hbm.at[g, pl.ds(wi * EW, EW)], w_v)

                def maskrow(r, _):
                    for k in range(8):
                        sl = pl.ds(k * 16, 16)
                        c = col_v[r, sl]
                        wv = w_v[r, sl]
                        inm = (c >= lo) & (c < lo + CH)
                        lcol_v[r, sl] = jnp.where(
                            inm, c - lo, dummy_base + (iota & 15))
                        weff_v[r, sl] = jnp.where(inm, wv, 0.0)
                    return 0

                lax.fori_loop(0, EW, maskrow, 0)

                gstart(0, rb0, sem_g0)

                def pair(j2, _):
                    j = 2 * j2
                    gstart(j + 1, rb1, sem_g1)
                    gwait(rb0, sem_g0)
                    process(j, rb0)

                    @pl.when(j2 < EW // 2 - 1)
                    def _():
                        gstart(j + 2, rb0, sem_g0)

                    gwait(rb1, sem_g1)
                    process(j + 1, rb1)
                    return 0

                lax.fori_loop(0, EW // 2, pair, 0)
                return 0

            lax.fori_loop(0, NWIN, window, 0)
            return 0

        lax.fori_loop(0, 2, group_body, 0)
        plsc.subcore_barrier()

        @pl.when(sid < NS - 1)
        def _():
            for off, nr in ((0, 128), (128, 128), (256, 128), (384, 128),
                            (512, 120)):
                pltpu.sync_copy(acc.at[pl.ds(sid * 632 + off, nr)],
                                rb0.at[pl.ds(0, nr)])
                pltpu.sync_copy(rb0.at[pl.ds(0, nr)],
                                out_hbm.at[pl.ds(lo + sid * 632 + off, nr)])

        @pl.when(sid == NS - 1)
        def _():
            for off, nr in ((0, 128), (128, 128), (256, 128), (384, 128),
                            (512, 8)):
                pltpu.sync_copy(acc.at[pl.ds(9480 + off, nr)],
                                rb0.at[pl.ds(0, nr)])
                pltpu.sync_copy(rb0.at[pl.ds(0, nr)],
                                out_hbm.at[pl.ds(lo + 9480 + off, nr)])

        plsc.subcore_barrier()
        return 0

    lax.fori_loop(0, 2, chunk_body, 0)


_sc_scatter = pl.kernel(
    _sc_scatter_body,
    out_type=jax.ShapeDtypeStruct((NTOT, IN_CH), jnp.float32),
    mesh=_mesh,
    scratch_types=[
        pltpu.VMEM((EW, 128), jnp.int32),
        pltpu.VMEM((EW, 128), jnp.int32),
        pltpu.VMEM((EW, 128), jnp.float32),
        pltpu.VMEM((EW, 128), jnp.int32),
        pltpu.VMEM((EW, 128), jnp.float32),
        pltpu.VMEM((GW, IN_CH), jnp.float32),
        pltpu.VMEM((GW, IN_CH), jnp.float32),
        pltpu.VMEM_SHARED((ACC_ROWS, IN_CH), jnp.float32),
        pltpu.SemaphoreType.DMA,
        pltpu.SemaphoreType.DMA,
        pltpu.SemaphoreType.DMA,
    ],
)


RB = 2000
NRB = NTOT // RB

_BN_S = 1.0 / (1.0 + EPS) ** 0.5


def _dinv_of(dref):
    d = dref[...]
    deg = d[0] + d[1] + 1.0
    return jnp.where(deg > 0, lax.rsqrt(deg), 0.0)


def _tc_prep_body(x_ref, w_ref, d_ref, xs_ref):
    dinv = _dinv_of(d_ref)
    xw = jnp.dot(x_ref[...], w_ref[...], preferred_element_type=jnp.float32)
    xs_ref[...] = dinv * xw


def _tc_combine_body(acc_ref, xs_ref, d_ref, b_ref, g_ref, be_ref, w2_ref,
                     x1_ref, xs2_ref):
    dinv = _dinv_of(d_ref)
    pre = dinv * (acc_ref[...] + xs_ref[...]) + b_ref[...]
    x1 = jnp.maximum(pre, 0.0) * (g_ref[...] * _BN_S) + be_ref[...]
    x1_ref[...] = x1
    xs2_ref[...] = dinv * jnp.dot(x1, w2_ref[...],
                                  preferred_element_type=jnp.float32)


def _tc_final_body(acc_ref, xs_ref, d_ref, b_ref, g_ref, be_ref, x2_ref):
    dinv = _dinv_of(d_ref)
    pre = dinv * (acc_ref[...] + xs_ref[...]) + b_ref[...]
    x2_ref[...] = jnp.maximum(pre, 0.0) * (g_ref[...] * _BN_S) + be_ref[...]


def _row_spec():
    return pl.BlockSpec((RB, 128), lambda i: (i, 0))


def _deg_spec():
    return pl.BlockSpec((NC, RB, 1), lambda i: (0, i, 0))


def _full(shape):
    return pl.BlockSpec(shape, lambda i: tuple(0 for _ in shape))


def _tc_prep(x, w, deg3):
    return pl.pallas_call(
        _tc_prep_body,
        grid=(NRB,),
        in_specs=[_row_spec(), _full((128, 128)), _deg_spec()],
        out_specs=_row_spec(),
        out_shape=jax.ShapeDtypeStruct((NTOT, 128), jnp.float32),
    )(x, w, deg3)


def _tc_combine(acc, xs, deg3, b, g, be, w2):
    return pl.pallas_call(
        _tc_combine_body,
        grid=(NRB,),
        in_specs=[_row_spec(), _row_spec(), _deg_spec(),
                  _full((128,)), _full((128,)), _full((128,)),
                  _full((128, 128))],
        out_specs=[_row_spec(), _row_spec()],
        out_shape=[jax.ShapeDtypeStruct((NTOT, 128), jnp.float32),
                   jax.ShapeDtypeStruct((NTOT, 128), jnp.float32)],
    )(acc, xs, deg3, b, g, be, w2)


def _tc_final(acc, xs, deg3, b, g, be):
    return pl.pallas_call(
        _tc_final_body,
        grid=(NRB,),
        in_specs=[_row_spec(), _row_spec(), _deg_spec(),
                  _full((128,)), _full((128,)), _full((128,))],
        out_specs=_row_spec(),
        out_shape=jax.ShapeDtypeStruct((NTOT, 128), jnp.float32),
    )(acc, xs, deg3, b, g, be)


LB = 2000
NLB = NUM_NODES // LB


def _lstm_body(x1_ref, x2_ref, a1_ref, b1w_ref, u1_ref, bias1_ref,
               wi2_ref, u2_ref, bias2_ref, h1_ref, h2_ref):
    x1 = x1_ref[...]
    x2 = x2_ref[...]
    a1 = a1_ref[...]
    b1w = b1w_ref[...]
    u1 = u1_ref[...]
    bias1 = bias1_ref[...]
    wi2 = wi2_ref[...]
    u2 = u2_ref[...]
    bias2 = bias2_ref[...]
    z = jnp.zeros((LB, HID), jnp.float32)
    h1 = c1 = h2 = c2 = z

    def cell(gates, c):
        i_ = jax.nn.sigmoid(gates[:, 0:HID])
        f_ = jax.nn.sigmoid(gates[:, HID:2 * HID])
        g_ = jnp.tanh(gates[:, 2 * HID:3 * HID])
        o_ = jax.nn.sigmoid(gates[:, 3 * HID:4 * HID])
        c = f_ * c + i_ * g_
        return o_ * jnp.tanh(c), c

    for t in range(WINDOW):
        g1 = (jnp.dot(x1[t], a1, preferred_element_type=jnp.float32)
              + jnp.dot(x2[t], b1w, preferred_element_type=jnp.float32)
              + jnp.dot(h1, u1, preferred_element_type=jnp.float32) + bias1)
        h1, c1 = cell(g1, c1)
        g2 = (jnp.dot(h1, wi2, preferred_element_type=jnp.float32)
              + jnp.dot(h2, u2, preferred_element_type=jnp.float32) + bias2)
        h2, c2 = cell(g2, c2)

    h1_ref[...] = h1
    h2_ref[...] = h2


def _tc_lstm(x1r, x2r, a1, b1w, u1, bias1, wi2, u2, bias2):
    blk = pl.BlockSpec((WINDOW, LB, 128), lambda i: (0, i, 0))
    out = pl.BlockSpec((LB, HID), lambda i: (i, 0))
    return pl.pallas_call(
        _lstm_body,
        grid=(NLB,),
        in_specs=[blk, blk, _full((128, 512)), _full((128, 512)),
                  _full((128, 512)), _full((512,)), _full((128, 512)),
                  _full((128, 512)), _full((512,))],
        out_specs=[out, out],
        out_shape=[jax.ShapeDtypeStruct((NUM_NODES, HID), jnp.float32),
                   jax.ShapeDtypeStruct((NUM_NODES, HID), jnp.float32)],
    )(x1r, x2r, a1, b1w, u1, bias1, wi2, u2, bias2)


def kernel(X, edge_index, edge_weight, W1, b1, g1, be1, W2, b2, g2, be2,
           Wih1, Whh1, bih1, bhh1, Wih2, Whh2, bih2, bhh2):
    row = edge_index[0]
    col = edge_index[1]
    pad = EPAD - E
    fill = (jnp.arange(pad, dtype=jnp.int32) * 977) % NTOT
    row_b = jnp.concatenate([row, fill]).reshape(NTILES, PT_ROWS, 128)
    col_b = jnp.concatenate([col, fill]).reshape(NTILES, PT_ROWS, 128)
    w_b = jnp.concatenate(
        [edge_weight, jnp.zeros((pad,), jnp.float32)]
    ).reshape(NTILES, PT_ROWS, 128)

    deg_part = _sc_deg(col_b, w_b)
    deg3 = deg_part.reshape(NC, NTOT, 1)

    xs1 = _tc_prep(X, W1, deg3)
    acc1 = _sc_scatter(xs1, row_b, col_b, w_b)
    X1, xs2 = _tc_combine(acc1, xs1, deg3, b1, g1, be1, W2)
    acc2 = _sc_scatter(xs2, row_b, col_b, w_b)
    X2 = _tc_final(acc2, xs2, deg3, b2, g2, be2)

    x1r = X1.reshape(WINDOW, NUM_NODES, 128)
    x2r = X2.reshape(WINDOW, NUM_NODES, 128)
    wt1 = Wih1.T
    a1 = wt1[:HID]
    b1w = wt1[HID:]
    H1, H2 = _tc_lstm(x1r, x2r, a1, b1w, Whh1.T, bih1 + bhh1,
                      Wih2.T, Whh2.T, bih2 + bhh2)

    s0 = X[0:NUM_NODES]
    scols = [X[l * NUM_NODES:(l + 1) * NUM_NODES, IN_CH - 1:IN_CH]
             for l in range(1, WINDOW)]
    return jnp.concatenate([H1, H2, s0] + scols, axis=1)
